# Initial kernel scaffold; baseline (speedup 1.0000x reference)
import jax
import jax.numpy as jnp
from jax.experimental import pallas as pl


def kernel(z1, z2, edge_index):
    raise NotImplementedError("write your pallas kernel here")


# speedup vs baseline: 1.1119x; 1.1119x over previous
import jax
import jax.numpy as jnp
from jax import lax
from jax.experimental import pallas as pl
from jax.experimental.pallas import tpu as pltpu
from jax.experimental.pallas import tpu_sc as plsc

_E = 320000
_D = 128

_info = plsc.get_sparse_core_info()
_NC, _NS, _L = _info.num_cores, _info.num_subcores, _info.num_lanes
_NW = _NC * _NS
_E_PER_W = _E // _NW
_CHUNK = 80
_NCHUNK = _E_PER_W // _CHUNK
_GRP = _CHUNK // _L


def _sc_body(z1_hbm, z2_hbm, src_hbm, dst_hbm, out_hbm,
             sidx, didx, srows, drows, outbuf, sem):
    wid = lax.axis_index("s") * _NC + lax.axis_index("c")
    base = wid * _E_PER_W
    pltpu.sync_copy(src_hbm.at[pl.ds(base, _E_PER_W)], sidx)
    pltpu.sync_copy(dst_hbm.at[pl.ds(base, _E_PER_W)], didx)
    lanes = lax.iota(jnp.int32, _L)

    def chunk_body(ci, carry):
        off = ci * _CHUNK
        pltpu.async_copy(z1_hbm.at[sidx.at[pl.ds(off, _CHUNK)]], srows, sem).wait()
        pltpu.async_copy(z2_hbm.at[didx.at[pl.ds(off, _CHUNK)]], drows, sem).wait()

        def grp_body(g, carry2):
            rows = g * _L + lanes
            acc = jnp.zeros((_L,), jnp.float32)
            for d in range(_D):
                cols = jnp.full((_L,), d, jnp.int32)
                s = plsc.load_gather(srows, [rows, cols])
                t = plsc.load_gather(drows, [rows, cols])
                acc = acc + s * t
            outbuf[pl.ds(off + g * _L, _L)] = 1.0 / (1.0 + jnp.exp(-acc))
            return carry2

        lax.fori_loop(0, _GRP, grp_body, 0)
        return carry

    lax.fori_loop(0, _NCHUNK, chunk_body, 0)
    pltpu.sync_copy(outbuf, out_hbm.at[pl.ds(base, _E_PER_W)])


def kernel(z1, z2, edge_index):
    mesh = plsc.VectorSubcoreMesh(core_axis_name="c", subcore_axis_name="s")
    k = pl.kernel(
        _sc_body,
        out_type=jax.ShapeDtypeStruct((_E,), jnp.float32),
        mesh=mesh,
        compiler_params=pltpu.CompilerParams(needs_layout_passes=False),
        scratch_types=[
            pltpu.VMEM((_E_PER_W,), jnp.int32),
            pltpu.VMEM((_E_PER_W,), jnp.int32),
            pltpu.VMEM((_CHUNK, _D), jnp.float32),
            pltpu.VMEM((_CHUNK, _D), jnp.float32),
            pltpu.VMEM((_E_PER_W,), jnp.float32),
            pltpu.SemaphoreType.DMA,
        ],
    )
    ei = edge_index.astype(jnp.int32)
    return k(z1, z2, ei[0], ei[1])


# speedup vs baseline: 1.3383x; 1.2036x over previous
import jax
import jax.numpy as jnp
from jax import lax
from jax.experimental import pallas as pl
from jax.experimental.pallas import tpu as pltpu
from jax.experimental.pallas import tpu_sc as plsc

_E = 320000
_D = 128

_info = plsc.get_sparse_core_info()
_NC, _NS, _L = _info.num_cores, _info.num_subcores, _info.num_lanes
_NW = _NC * _NS
_E_PER_W = _E // _NW
_CHUNK = 80
_NCHUNK = _E_PER_W // _CHUNK
_GRP = _CHUNK // _L


def _sc_body(z1_hbm, z2_hbm, src_hbm, dst_hbm, out_hbm,
             sidx, didx, sr_a, dr_a, sr_b, dr_b, outbuf, sem_a, sem_b):
    wid = lax.axis_index("s") * _NC + lax.axis_index("c")
    base = wid * _E_PER_W
    pltpu.sync_copy(src_hbm.at[pl.ds(base, _E_PER_W)], sidx)
    pltpu.sync_copy(dst_hbm.at[pl.ds(base, _E_PER_W)], didx)
    lanes = lax.iota(jnp.int32, _L)

    def start(ci, srows, drows, sem):
        off = ci * _CHUNK
        pltpu.async_copy(z1_hbm.at[sidx.at[pl.ds(off, _CHUNK)]], srows, sem)
        pltpu.async_copy(z2_hbm.at[didx.at[pl.ds(off, _CHUNK)]], drows, sem)

    def wait(ci, srows, drows, sem):
        off = ci * _CHUNK
        pltpu.make_async_copy(
            z1_hbm.at[sidx.at[pl.ds(off, _CHUNK)]], srows, sem).wait()
        pltpu.make_async_copy(
            z2_hbm.at[didx.at[pl.ds(off, _CHUNK)]], drows, sem).wait()

    def compute(ci, srows, drows):
        off = ci * _CHUNK

        def grp_body(g, carry2):
            rows = g * _L + lanes
            acc = jnp.zeros((_L,), jnp.float32)
            for d in range(_D):
                cols = jnp.full((_L,), d, jnp.int32)
                s = plsc.load_gather(srows, [rows, cols])
                t = plsc.load_gather(drows, [rows, cols])
                acc = acc + s * t
            outbuf[pl.ds(off + g * _L, _L)] = 1.0 / (1.0 + jnp.exp(-acc))
            return carry2

        lax.fori_loop(0, _GRP, grp_body, 0)

    start(0, sr_a, dr_a, sem_a)

    def chunk_body(ci, carry):
        @pl.when(ci % 2 == 0)
        def _even():
            @pl.when(ci + 1 < _NCHUNK)
            def _():
                start(ci + 1, sr_b, dr_b, sem_b)
            wait(ci, sr_a, dr_a, sem_a)
            compute(ci, sr_a, dr_a)

        @pl.when(ci % 2 == 1)
        def _odd():
            @pl.when(ci + 1 < _NCHUNK)
            def _():
                start(ci + 1, sr_a, dr_a, sem_a)
            wait(ci, sr_b, dr_b, sem_b)
            compute(ci, sr_b, dr_b)

        return carry

    lax.fori_loop(0, _NCHUNK, chunk_body, 0)
    pltpu.sync_copy(outbuf, out_hbm.at[pl.ds(base, _E_PER_W)])


def kernel(z1, z2, edge_index):
    mesh = plsc.VectorSubcoreMesh(core_axis_name="c", subcore_axis_name="s")
    k = pl.kernel(
        _sc_body,
        out_type=jax.ShapeDtypeStruct((_E,), jnp.float32),
        mesh=mesh,
        compiler_params=pltpu.CompilerParams(needs_layout_passes=False),
        scratch_types=[
            pltpu.VMEM((_E_PER_W,), jnp.int32),
            pltpu.VMEM((_E_PER_W,), jnp.int32),
            pltpu.VMEM((_CHUNK, _D), jnp.float32),
            pltpu.VMEM((_CHUNK, _D), jnp.float32),
            pltpu.VMEM((_CHUNK, _D), jnp.float32),
            pltpu.VMEM((_CHUNK, _D), jnp.float32),
            pltpu.VMEM((_E_PER_W,), jnp.float32),
            pltpu.SemaphoreType.DMA,
            pltpu.SemaphoreType.DMA,
        ],
    )
    ei = edge_index.astype(jnp.int32)
    return k(z1, z2, ei[0], ei[1])
